# baseline (speedup 1.0000x reference)
import jax
import jax.numpy as jnp
from jax import lax
from jax.experimental import pallas as pl
from jax.experimental.pallas import tpu as pltpu
from jax.experimental.pallas import tpu_sc as plsc

_NC = 2
_NS = 16
_NW = _NC * _NS
_CHUNK = 128
_G = 8


def _pad_rows(N):
    return -(-N // (_NS * 8)) * (_NS * 8)


def _chunks_per_worker(E):
    return -(-E // (_NW * _CHUNK * _G)) * _G


def _sc_agg_kernel(N, E, D, with_deg):
    n_chunks = _chunks_per_worker(E)
    n_groups = n_chunks // _G
    NP = _pad_rows(N)
    RW = NP // _NS
    assert n_groups % 2 == 0

    mesh = plsc.VectorSubcoreMesh(core_axis_name="c", subcore_axis_name="s")

    out_type = [jax.ShapeDtypeStruct((_NC, NP, D), jnp.float32)]
    if with_deg:
        out_type.append(jax.ShapeDtypeStruct((_NC, NP, 128), jnp.float32))

    scratch = [
        pltpu.VMEM((_G, _CHUNK), jnp.int32),
        pltpu.VMEM((_G, _CHUNK), jnp.int32),
        pltpu.VMEM((_G, _CHUNK), jnp.int32),
        pltpu.VMEM((_G, _CHUNK), jnp.int32),
        pltpu.VMEM((_CHUNK, D), jnp.float32),
        pltpu.VMEM((_CHUNK, D), jnp.float32),
        pltpu.VMEM_SHARED((NP, D), jnp.float32),
        pltpu.SemaphoreType.DMA,
        pltpu.SemaphoreType.DMA,
        pltpu.SemaphoreType.DMA,
        pltpu.SemaphoreType.DMA,
    ]

    n_zcopy = RW // _CHUNK
    z_tail = RW - n_zcopy * _CHUNK

    def body(*refs):
        if with_deg:
            (table, srci, dsti, zeros_h, ones_h,
             out_acc, out_deg,
             sg0, sg1, dg0, dg1, rows0, rows1, acc_sh,
             sem0, sem1, semi0, semi1) = refs
        else:
            (table, srci, dsti, zeros_h,
             out_acc,
             sg0, sg1, dg0, dg1, rows0, rows1, acc_sh,
             sem0, sem1, semi0, semi1) = refs
        c = lax.axis_index("c")
        s = lax.axis_index("s")
        w = c * _NS + s
        rows = (rows0, rows1)
        sems = (sem0, sem1)

        def zero_acc():
            base = s * RW
            for k in range(n_zcopy):
                pltpu.sync_copy(rows1,
                                acc_sh.at[pl.ds(base + k * _CHUNK, _CHUNK)])
            if z_tail:
                pltpu.sync_copy(
                    rows1.at[pl.ds(0, z_tail)],
                    acc_sh.at[pl.ds(base + n_zcopy * _CHUNK, z_tail)])

        def load_idx_group(g, sg, dg, semi):
            pltpu.async_copy(srci.at[w, pl.ds(g * _G, _G)], sg, semi)
            pltpu.async_copy(dsti.at[w, pl.ds(g * _G, _G)], dg, semi)

        def wait_idx_group(sg, dg, semi):
            pltpu.make_async_copy(srci.at[w, pl.ds(0, _G)], sg, semi).wait()
            pltpu.make_async_copy(dsti.at[w, pl.ds(0, _G)], dg, semi).wait()

        def wait_rows(buf, sem):
            pltpu.make_async_copy(table.at[pl.ds(0, _CHUNK)], buf, sem).wait()

        pltpu.sync_copy(zeros_h, rows1)

        if with_deg:
            pltpu.sync_copy(ones_h, rows0)
            zero_acc()
            plsc.subcore_barrier()

            def deg_group(g, carry):
                pltpu.sync_copy(dsti.at[w, pl.ds(g * _G, _G)], dg0)
                for j in range(_G):
                    pltpu.sync_copy(rows0, acc_sh.at[dg0.at[j]], add=True)
                return carry

            lax.fori_loop(0, n_groups, deg_group, 0)
            plsc.subcore_barrier()
            pltpu.sync_copy(acc_sh.at[pl.ds(s * RW, RW)],
                            out_deg.at[c, pl.ds(s * RW, RW)])
            plsc.subcore_barrier()

        zero_acc()
        plsc.subcore_barrier()

        pltpu.sync_copy(srci.at[w, pl.ds(0, _G)], sg0)
        pltpu.sync_copy(dsti.at[w, pl.ds(0, _G)], dg0)
        load_idx_group(1, sg1, dg1, semi1)

        def do_group(sg, dg):
            for j in range(_G):
                pltpu.async_copy(table.at[sg.at[j]], rows0, sem0)
                wait_rows(rows0, sem0)
                pltpu.sync_copy(rows0, acc_sh.at[dg.at[j]], add=True)

        def group_pair(t, carry):
            g0 = 2 * t

            @pl.when(t > 0)
            def _():
                wait_idx_group(sg0, dg0, semi0)

            do_group(sg0, dg0)

            @pl.when(g0 + 2 < n_groups)
            def _():
                load_idx_group(g0 + 2, sg0, dg0, semi0)

            wait_idx_group(sg1, dg1, semi1)
            do_group(sg1, dg1)

            @pl.when(g0 + 3 < n_groups)
            def _():
                load_idx_group(g0 + 3, sg1, dg1, semi1)

            return carry

        lax.fori_loop(0, n_groups // 2, group_pair, 0)
        plsc.subcore_barrier()

        pltpu.sync_copy(acc_sh.at[pl.ds(s * RW, RW)],
                        out_acc.at[c, pl.ds(s * RW, RW)])

    return pl.kernel(body, out_type=out_type, mesh=mesh, scratch_types=scratch)


def _tc_layer(N, D, DO, relu, with_deg):
    R = 2000
    grid = (N // R,)

    def body(*refs):
        if with_deg:
            (x_r, a0_r, a1_r, d0_r, d1_r, ws_r, wn_r, b_r, o_r, inv_r) = refs
            deg = d0_r[:, 0:1] + d1_r[:, 0:1]
            inv = 1.0 / jnp.maximum(deg, 1.0)
            inv_r[...] = inv
        else:
            (x_r, a0_r, a1_r, inv_in_r, ws_r, wn_r, b_r, o_r) = refs
            inv = inv_in_r[...]
        hn = (a0_r[...] + a1_r[...]) * inv
        acc = jnp.dot(x_r[...], ws_r[...], preferred_element_type=jnp.float32)
        acc = acc + jnp.dot(hn, wn_r[...], preferred_element_type=jnp.float32)
        acc = acc + b_r[...]
        if relu:
            acc = jnp.maximum(acc, 0.0)
        o_r[...] = acc

    def row_blk(d):
        return pl.BlockSpec((R, d), lambda i: (i, 0))

    def full(shape):
        return pl.BlockSpec(shape, lambda i: (0,) * len(shape))

    if with_deg:
        in_specs = [row_blk(D), row_blk(D), row_blk(D), row_blk(128),
                    row_blk(128),
                    full((D, DO)), full((D, DO)), full((1, DO))]
        out_shape = [jax.ShapeDtypeStruct((N, DO), jnp.float32),
                     jax.ShapeDtypeStruct((N, 1), jnp.float32)]
        out_specs = [row_blk(DO), row_blk(1)]
    else:
        in_specs = [row_blk(D), row_blk(D), row_blk(D), row_blk(1),
                    full((D, DO)), full((D, DO)), full((1, DO))]
        out_shape = jax.ShapeDtypeStruct((N, DO), jnp.float32)
        out_specs = row_blk(DO)

    return pl.pallas_call(
        body, grid=grid, in_specs=in_specs, out_specs=out_specs,
        out_shape=out_shape)


def kernel(x, edge_index, W_self1, W_neigh1, b1, W_self2, W_neigh2, b2):
    N, D = x.shape
    E = edge_index.shape[1]
    n_cls = W_self2.shape[1]
    NP = _pad_rows(N)

    n_chunks = _chunks_per_worker(E)
    EWP = n_chunks * _CHUNK
    EW = E // _NW
    assert EW * _NW == E
    src = edge_index[0].astype(jnp.int32).reshape(_NW, EW)
    dst = edge_index[1].astype(jnp.int32).reshape(_NW, EW)
    pad = EWP - EW
    src = jnp.pad(src, ((0, 0), (0, pad))).reshape(_NW, n_chunks, _CHUNK)
    dst = jnp.pad(dst, ((0, 0), (0, pad)), constant_values=NP - 1)
    dst = dst.reshape(_NW, n_chunks, _CHUNK)

    sc1 = _sc_agg_kernel(N, E, D, with_deg=True)
    sc2 = _sc_agg_kernel(N, E, D, with_deg=False)
    tc1 = _tc_layer(N, D, D, relu=True, with_deg=True)
    tc2 = _tc_layer(N, D, D, relu=False, with_deg=False)

    zeros_h = jnp.zeros((_CHUNK, D), jnp.float32)
    ones_h = jnp.ones((_CHUNK, 128), jnp.float32)
    acc1, degp = sc1(x, src, dst, zeros_h, ones_h)
    h, inv = tc1(x, acc1[0], acc1[1], degp[0], degp[1],
                 W_self1, W_neigh1, b1.reshape(1, D))

    acc2 = sc2(h, src, dst, zeros_h)
    if isinstance(acc2, (list, tuple)):
        acc2 = acc2[0]
    W_self2p = jnp.pad(W_self2, ((0, 0), (0, D - n_cls)))
    W_neigh2p = jnp.pad(W_neigh2, ((0, 0), (0, D - n_cls)))
    b2p = jnp.pad(b2, (0, D - n_cls)).reshape(1, D)
    outp = tc2(h, acc2[0], acc2[1], inv, W_self2p, W_neigh2p, b2p)
    return outp[:, :n_cls]


# speedup vs baseline: 2.4455x; 2.4455x over previous
import jax
import jax.numpy as jnp
from jax import lax
from jax.experimental import pallas as pl
from jax.experimental.pallas import tpu as pltpu
from jax.experimental.pallas import tpu_sc as plsc

_NC = 2
_NS = 16
_NW = _NC * _NS
_CHUNK = 80


def _pad_rows(N):
    return -(-N // (_NS * 8)) * (_NS * 8)


def _sc_agg_kernel(N, E, D, with_deg):
    EW = E // _NW
    n = EW // _CHUNK
    NP = _pad_rows(N)
    RW = NP // _NS
    assert EW * _NW == E and n * _CHUNK == EW

    mesh = plsc.VectorSubcoreMesh(core_axis_name="c", subcore_axis_name="s")

    out_type = [jax.ShapeDtypeStruct((_NC, NP, D), jnp.float32)]
    if with_deg:
        out_type.append(jax.ShapeDtypeStruct((_NC, NP, 128), jnp.float32))

    scratch = [
        pltpu.VMEM((_CHUNK,), jnp.int32),
        pltpu.VMEM((_CHUNK,), jnp.int32),
        pltpu.VMEM((_CHUNK,), jnp.int32),
        pltpu.VMEM((_CHUNK,), jnp.int32),
        pltpu.VMEM((_CHUNK, D), jnp.float32),
        pltpu.VMEM((_CHUNK, D), jnp.float32),
        pltpu.VMEM_SHARED((NP, D), jnp.float32),
        pltpu.SemaphoreType.DMA,
        pltpu.SemaphoreType.DMA,
        pltpu.SemaphoreType.DMA,
        pltpu.SemaphoreType.DMA,
    ]

    n_zcopy = RW // _CHUNK
    z_tail = RW - n_zcopy * _CHUNK

    def body(*refs):
        if with_deg:
            (table, srci, dsti, zeros_h, ones_h,
             out_acc, out_deg,
             sa0, sa1, da0, da1, rows0, rows1, acc_sh,
             sem0, sem1, semi0, semi1) = refs
        else:
            (table, srci, dsti, zeros_h,
             out_acc,
             sa0, sa1, da0, da1, rows0, rows1, acc_sh,
             sem0, sem1, semi0, semi1) = refs
        c = lax.axis_index("c")
        s = lax.axis_index("s")
        w = c * _NS + s

        def off(i):
            return w * EW + i * _CHUNK

        def zero_acc():
            base = s * RW
            for k in range(n_zcopy):
                pltpu.sync_copy(rows1,
                                acc_sh.at[pl.ds(base + k * _CHUNK, _CHUNK)])
            if z_tail:
                pltpu.sync_copy(
                    rows1.at[pl.ds(0, z_tail)],
                    acc_sh.at[pl.ds(base + n_zcopy * _CHUNK, z_tail)])

        def load_pair(i, sa, da, semi):
            pltpu.async_copy(srci.at[pl.ds(off(i), _CHUNK)], sa, semi)
            pltpu.async_copy(dsti.at[pl.ds(off(i), _CHUNK)], da, semi)

        def wait_pair(sa, da, semi):
            pltpu.make_async_copy(srci.at[pl.ds(0, _CHUNK)], sa, semi).wait()
            pltpu.make_async_copy(dsti.at[pl.ds(0, _CHUNK)], da, semi).wait()

        def load_dst(i, da, semi):
            pltpu.async_copy(dsti.at[pl.ds(off(i), _CHUNK)], da, semi)

        def wait_dst(da, semi):
            pltpu.make_async_copy(dsti.at[pl.ds(0, _CHUNK)], da, semi).wait()

        def wait_rows(buf, sem):
            pltpu.make_async_copy(table.at[pl.ds(0, _CHUNK)], buf, sem).wait()

        pltpu.sync_copy(zeros_h, rows1)

        if with_deg:
            pltpu.sync_copy(ones_h, rows0)
            zero_acc()
            plsc.subcore_barrier()

            pltpu.sync_copy(dsti.at[pl.ds(off(0), _CHUNK)], da0)
            load_dst(1, da1, semi1)

            def deg_pair(t, carry):
                i0 = 2 * t

                @pl.when(t > 0)
                def _():
                    wait_dst(da0, semi0)

                pltpu.sync_copy(rows0, acc_sh.at[da0], add=True)

                @pl.when(i0 + 2 < n)
                def _():
                    load_dst(i0 + 2, da0, semi0)

                @pl.when(i0 + 1 < n)
                def _():
                    wait_dst(da1, semi1)
                    pltpu.sync_copy(rows0, acc_sh.at[da1], add=True)

                    @pl.when(i0 + 3 < n)
                    def _():
                        load_dst(i0 + 3, da1, semi1)

                return carry

            lax.fori_loop(0, (n + 1) // 2, deg_pair, 0)
            plsc.subcore_barrier()
            pltpu.sync_copy(acc_sh.at[pl.ds(s * RW, RW)],
                            out_deg.at[c, pl.ds(s * RW, RW)])
            plsc.subcore_barrier()

        zero_acc()
        plsc.subcore_barrier()

        pltpu.sync_copy(srci.at[pl.ds(off(0), _CHUNK)], sa0)
        pltpu.sync_copy(dsti.at[pl.ds(off(0), _CHUNK)], da0)
        load_pair(1, sa1, da1, semi1)
        pltpu.async_copy(table.at[sa0], rows0, sem0)

        def feat_pair(t, carry):
            i0 = 2 * t

            @pl.when(i0 + 1 < n)
            def _():
                wait_pair(sa1, da1, semi1)
                pltpu.async_copy(table.at[sa1], rows1, sem1)

            wait_rows(rows0, sem0)
            pltpu.sync_copy(rows0, acc_sh.at[da0], add=True)

            @pl.when(i0 + 2 < n)
            def _():
                load_pair(i0 + 2, sa0, da0, semi0)
                wait_pair(sa0, da0, semi0)
                pltpu.async_copy(table.at[sa0], rows0, sem0)

            @pl.when(i0 + 1 < n)
            def _():
                wait_rows(rows1, sem1)
                pltpu.sync_copy(rows1, acc_sh.at[da1], add=True)

                @pl.when(i0 + 3 < n)
                def _():
                    load_pair(i0 + 3, sa1, da1, semi1)

            return carry

        lax.fori_loop(0, (n + 1) // 2, feat_pair, 0)
        plsc.subcore_barrier()

        pltpu.sync_copy(acc_sh.at[pl.ds(s * RW, RW)],
                        out_acc.at[c, pl.ds(s * RW, RW)])

    return pl.kernel(body, out_type=out_type, mesh=mesh, scratch_types=scratch)


def _tc_layer(N, D, DO, relu, with_deg):
    R = 2000
    grid = (N // R,)

    def body(*refs):
        if with_deg:
            (x_r, a0_r, a1_r, d0_r, d1_r, ws_r, wn_r, b_r, o_r, inv_r) = refs
            deg = d0_r[:, 0:1] + d1_r[:, 0:1]
            inv = 1.0 / jnp.maximum(deg, 1.0)
            inv_r[...] = inv
        else:
            (x_r, a0_r, a1_r, inv_in_r, ws_r, wn_r, b_r, o_r) = refs
            inv = inv_in_r[...]
        hn = (a0_r[...] + a1_r[...]) * inv
        acc = jnp.dot(x_r[...], ws_r[...], preferred_element_type=jnp.float32)
        acc = acc + jnp.dot(hn, wn_r[...], preferred_element_type=jnp.float32)
        acc = acc + b_r[...]
        if relu:
            acc = jnp.maximum(acc, 0.0)
        o_r[...] = acc

    def row_blk(d):
        return pl.BlockSpec((R, d), lambda i: (i, 0))

    def full(shape):
        return pl.BlockSpec(shape, lambda i: (0,) * len(shape))

    if with_deg:
        in_specs = [row_blk(D), row_blk(D), row_blk(D), row_blk(128),
                    row_blk(128),
                    full((D, DO)), full((D, DO)), full((1, DO))]
        out_shape = [jax.ShapeDtypeStruct((N, DO), jnp.float32),
                     jax.ShapeDtypeStruct((N, 1), jnp.float32)]
        out_specs = [row_blk(DO), row_blk(1)]
    else:
        in_specs = [row_blk(D), row_blk(D), row_blk(D), row_blk(1),
                    full((D, DO)), full((D, DO)), full((1, DO))]
        out_shape = jax.ShapeDtypeStruct((N, DO), jnp.float32)
        out_specs = row_blk(DO)

    return pl.pallas_call(
        body, grid=grid, in_specs=in_specs, out_specs=out_specs,
        out_shape=out_shape)


def kernel(x, edge_index, W_self1, W_neigh1, b1, W_self2, W_neigh2, b2):
    N, D = x.shape
    E = edge_index.shape[1]
    n_cls = W_self2.shape[1]

    src = edge_index[0].astype(jnp.int32)
    dst = edge_index[1].astype(jnp.int32)

    zeros_h = jnp.zeros((_CHUNK, D), jnp.float32)
    ones_h = jnp.ones((_CHUNK, 128), jnp.float32)

    sc1 = _sc_agg_kernel(N, E, D, with_deg=True)
    sc2 = _sc_agg_kernel(N, E, D, with_deg=False)
    tc1 = _tc_layer(N, D, D, relu=True, with_deg=True)
    tc2 = _tc_layer(N, D, D, relu=False, with_deg=False)

    acc1, degp = sc1(x, src, dst, zeros_h, ones_h)
    h, inv = tc1(x, acc1[0], acc1[1], degp[0], degp[1],
                 W_self1, W_neigh1, b1.reshape(1, D))

    acc2 = sc2(h, src, dst, zeros_h)
    if isinstance(acc2, (list, tuple)):
        acc2 = acc2[0]
    W_self2p = jnp.pad(W_self2, ((0, 0), (0, D - n_cls)))
    W_neigh2p = jnp.pad(W_neigh2, ((0, 0), (0, D - n_cls)))
    b2p = jnp.pad(b2, (0, D - n_cls)).reshape(1, D)
    outp = tc2(h, acc2[0], acc2[1], inv, W_self2p, W_neigh2p, b2p)
    return outp[:, :n_cls]


# speedup vs baseline: 2.8061x; 1.1474x over previous
import jax
import jax.numpy as jnp
from jax import lax
from jax.experimental import pallas as pl
from jax.experimental.pallas import tpu as pltpu
from jax.experimental.pallas import tpu_sc as plsc

_NC = 2
_NS = 16
_NW = _NC * _NS
_CHUNK = 80


def _pad_rows(N):
    return -(-N // (_NS * 8)) * (_NS * 8)


def _sc_agg_kernel(N, E, D, with_deg):
    EW = E // _NW
    n = EW // _CHUNK
    NP = _pad_rows(N)
    RW = NP // _NS
    assert EW * _NW == E and n * _CHUNK == EW

    mesh = plsc.VectorSubcoreMesh(core_axis_name="c", subcore_axis_name="s")

    out_type = [jax.ShapeDtypeStruct((_NC, NP, D), jnp.float32)]
    if with_deg:
        out_type.append(jax.ShapeDtypeStruct((_NC, NP, 128), jnp.float32))

    scratch = [
        pltpu.VMEM((_CHUNK,), jnp.int32),
        pltpu.VMEM((_CHUNK,), jnp.int32),
        pltpu.VMEM((_CHUNK,), jnp.int32),
        pltpu.VMEM((_CHUNK,), jnp.int32),
        pltpu.VMEM((_CHUNK,), jnp.int32),
        pltpu.VMEM((_CHUNK,), jnp.int32),
        pltpu.VMEM((_CHUNK, D), jnp.float32),
        pltpu.VMEM((_CHUNK, D), jnp.float32),
        pltpu.VMEM_SHARED((NP, D), jnp.float32),
        pltpu.SemaphoreType.DMA,
        pltpu.SemaphoreType.DMA,
        pltpu.SemaphoreType.DMA,
        pltpu.SemaphoreType.DMA,
        pltpu.SemaphoreType.DMA,
    ]

    n_zcopy = RW // _CHUNK
    z_tail = RW - n_zcopy * _CHUNK

    def body(*refs):
        if with_deg:
            (table, srci, dsti, zeros_h, ones_h,
             out_acc, out_deg,
             sa0, sa1, sa2, da0, da1, da2, rows0, rows1, acc_sh,
             sem0, sem1, semi0, semi1, semi2) = refs
        else:
            (table, srci, dsti, zeros_h,
             out_acc,
             sa0, sa1, sa2, da0, da1, da2, rows0, rows1, acc_sh,
             sem0, sem1, semi0, semi1, semi2) = refs
        c = lax.axis_index("c")
        s = lax.axis_index("s")
        w = c * _NS + s

        def off(i):
            return w * EW + i * _CHUNK

        def zero_acc():
            base = s * RW
            for k in range(n_zcopy):
                pltpu.sync_copy(rows1,
                                acc_sh.at[pl.ds(base + k * _CHUNK, _CHUNK)])
            if z_tail:
                pltpu.sync_copy(
                    rows1.at[pl.ds(0, z_tail)],
                    acc_sh.at[pl.ds(base + n_zcopy * _CHUNK, z_tail)])

        def load_pair(i, sa, da, semi):
            pltpu.async_copy(srci.at[pl.ds(off(i), _CHUNK)], sa, semi)
            pltpu.async_copy(dsti.at[pl.ds(off(i), _CHUNK)], da, semi)

        def wait_pair(sa, da, semi):
            pltpu.make_async_copy(srci.at[pl.ds(0, _CHUNK)], sa, semi).wait()
            pltpu.make_async_copy(dsti.at[pl.ds(0, _CHUNK)], da, semi).wait()

        def load_dst(i, da, semi):
            pltpu.async_copy(dsti.at[pl.ds(off(i), _CHUNK)], da, semi)

        def wait_dst(da, semi):
            pltpu.make_async_copy(dsti.at[pl.ds(0, _CHUNK)], da, semi).wait()

        def wait_rows(buf, sem):
            pltpu.make_async_copy(table.at[pl.ds(0, _CHUNK)], buf, sem).wait()

        pltpu.sync_copy(zeros_h, rows1)

        if with_deg:
            pltpu.sync_copy(ones_h, rows0)
            zero_acc()
            plsc.subcore_barrier()

            pltpu.sync_copy(dsti.at[pl.ds(off(0), _CHUNK)], da0)
            load_dst(1, da1, semi1)

            def deg_pair(t, carry):
                i0 = 2 * t

                @pl.when(t > 0)
                def _():
                    wait_dst(da0, semi0)

                pltpu.sync_copy(rows0, acc_sh.at[da0], add=True)

                @pl.when(i0 + 2 < n)
                def _():
                    load_dst(i0 + 2, da0, semi0)

                @pl.when(i0 + 1 < n)
                def _():
                    wait_dst(da1, semi1)
                    pltpu.sync_copy(rows0, acc_sh.at[da1], add=True)

                    @pl.when(i0 + 3 < n)
                    def _():
                        load_dst(i0 + 3, da1, semi1)

                return carry

            lax.fori_loop(0, (n + 1) // 2, deg_pair, 0)
            plsc.subcore_barrier()
            pltpu.sync_copy(acc_sh.at[pl.ds(s * RW, RW)],
                            out_deg.at[c, pl.ds(s * RW, RW)])
            plsc.subcore_barrier()

        zero_acc()
        plsc.subcore_barrier()

        pltpu.sync_copy(srci.at[pl.ds(off(0), _CHUNK)], sa0)
        pltpu.sync_copy(dsti.at[pl.ds(off(0), _CHUNK)], da0)
        load_pair(1, sa1, da1, semi1)
        load_pair(2, sa2, da2, semi2)
        pltpu.async_copy(table.at[sa0], rows0, sem0)

        sas = (sa0, sa1, sa2)
        das = (da0, da1, da2)
        semis = (semi0, semi1, semi2)
        rr = (rows0, rows1)
        ss = (sem0, sem1)

        def feat_six(u, carry):
            base = 6 * u
            for j in range(6):
                i = base + j
                p3 = j % 3
                q3 = (j + 1) % 3
                b = j % 2

                @pl.when(i < n)
                def _(i=i, p3=p3, q3=q3, b=b):
                    @pl.when(i + 1 < n)
                    def _():
                        wait_pair(sas[q3], das[q3], semis[q3])
                        pltpu.async_copy(table.at[sas[q3]],
                                         rr[1 - b], ss[1 - b])

                    wait_rows(rr[b], ss[b])
                    pltpu.sync_copy(rr[b], acc_sh.at[das[p3]], add=True)

                    @pl.when(i + 3 < n)
                    def _():
                        load_pair(i + 3, sas[p3], das[p3], semis[p3])

            return carry

        lax.fori_loop(0, (n + 5) // 6, feat_six, 0)
        plsc.subcore_barrier()

        pltpu.sync_copy(acc_sh.at[pl.ds(s * RW, RW)],
                        out_acc.at[c, pl.ds(s * RW, RW)])

    return pl.kernel(body, out_type=out_type, mesh=mesh, scratch_types=scratch)


def _tc_layer(N, D, DO, relu, with_deg):
    R = 2000
    grid = (N // R,)

    def body(*refs):
        if with_deg:
            (x_r, a0_r, a1_r, d0_r, d1_r, ws_r, wn_r, b_r, o_r, inv_r) = refs
            deg = d0_r[:, 0:1] + d1_r[:, 0:1]
            inv = 1.0 / jnp.maximum(deg, 1.0)
            inv_r[...] = inv
        else:
            (x_r, a0_r, a1_r, inv_in_r, ws_r, wn_r, b_r, o_r) = refs
            inv = inv_in_r[...]
        hn = (a0_r[...] + a1_r[...]) * inv
        acc = jnp.dot(x_r[...], ws_r[...], preferred_element_type=jnp.float32)
        acc = acc + jnp.dot(hn, wn_r[...], preferred_element_type=jnp.float32)
        acc = acc + b_r[...]
        if relu:
            acc = jnp.maximum(acc, 0.0)
        o_r[...] = acc

    def row_blk(d):
        return pl.BlockSpec((R, d), lambda i: (i, 0))

    def full(shape):
        return pl.BlockSpec(shape, lambda i: (0,) * len(shape))

    if with_deg:
        in_specs = [row_blk(D), row_blk(D), row_blk(D), row_blk(128),
                    row_blk(128),
                    full((D, DO)), full((D, DO)), full((1, DO))]
        out_shape = [jax.ShapeDtypeStruct((N, DO), jnp.float32),
                     jax.ShapeDtypeStruct((N, 1), jnp.float32)]
        out_specs = [row_blk(DO), row_blk(1)]
    else:
        in_specs = [row_blk(D), row_blk(D), row_blk(D), row_blk(1),
                    full((D, DO)), full((D, DO)), full((1, DO))]
        out_shape = jax.ShapeDtypeStruct((N, DO), jnp.float32)
        out_specs = row_blk(DO)

    return pl.pallas_call(
        body, grid=grid, in_specs=in_specs, out_specs=out_specs,
        out_shape=out_shape)


def kernel(x, edge_index, W_self1, W_neigh1, b1, W_self2, W_neigh2, b2):
    N, D = x.shape
    E = edge_index.shape[1]
    n_cls = W_self2.shape[1]

    src = edge_index[0].astype(jnp.int32)
    dst = edge_index[1].astype(jnp.int32)

    zeros_h = jnp.zeros((_CHUNK, D), jnp.float32)
    ones_h = jnp.ones((_CHUNK, 128), jnp.float32)

    sc1 = _sc_agg_kernel(N, E, D, with_deg=True)
    sc2 = _sc_agg_kernel(N, E, D, with_deg=False)
    tc1 = _tc_layer(N, D, D, relu=True, with_deg=True)
    tc2 = _tc_layer(N, D, D, relu=False, with_deg=False)

    acc1, degp = sc1(x, src, dst, zeros_h, ones_h)
    h, inv = tc1(x, acc1[0], acc1[1], degp[0], degp[1],
                 W_self1, W_neigh1, b1.reshape(1, D))

    acc2 = sc2(h, src, dst, zeros_h)
    if isinstance(acc2, (list, tuple)):
        acc2 = acc2[0]
    W_self2p = jnp.pad(W_self2, ((0, 0), (0, D - n_cls)))
    W_neigh2p = jnp.pad(W_neigh2, ((0, 0), (0, D - n_cls)))
    b2p = jnp.pad(b2, (0, D - n_cls)).reshape(1, D)
    outp = tc2(h, acc2[0], acc2[1], inv, W_self2p, W_neigh2p, b2p)
    return outp[:, :n_cls]
